# baseline (speedup 1.0000x reference)
import functools

import jax
import jax.numpy as jnp
from jax import lax
from jax.experimental import pallas as pl
from jax.experimental.pallas import tpu as pltpu
from jax.experimental.pallas import tpu_sc as plsc

N = 50000
C = 128
K = 9
EPS = 1e-5

NC = 2
NS = 16
NW = NC * NS
L = 16
NG = C // L

CHUNK = 56
NCHUNK0 = 32
NCHUNK1 = 24
TOTCHUNK = NS * (NCHUNK0 + NCHUNK1)
NPAD = CHUNK * TOTCHUNK
IDXROWS = 288

MM_BLK = 1000
N_BLOCKS = N // MM_BLK

_mesh = plsc.VectorSubcoreMesh(
    core_axis_name="c", subcore_axis_name="s", num_cores=NC, num_subcores=NS)


@functools.partial(
    pl.kernel,
    out_type=(
        jax.ShapeDtypeStruct((NPAD, C), jnp.float32),
        jax.ShapeDtypeStruct((NW, 2, C), jnp.float32),
    ),
    mesh=_mesh,
    scratch_types=[
        pltpu.VMEM((IDXROWS, CHUNK), jnp.int32),
        pltpu.VMEM((CHUNK, C), jnp.float32),
        pltpu.VMEM((CHUNK, C), jnp.float32),
        pltpu.VMEM((CHUNK, C), jnp.float32),
        pltpu.VMEM((CHUNK, C), jnp.float32),
        pltpu.VMEM((CHUNK, C), jnp.float32),
        pltpu.VMEM((CHUNK, C), jnp.float32),
        pltpu.VMEM((CHUNK, C), jnp.float32),
        pltpu.VMEM((CHUNK, C), jnp.float32),
        pltpu.VMEM((CHUNK, C), jnp.float32),
        pltpu.VMEM((CHUNK, C), jnp.float32),
        pltpu.VMEM((2, C), jnp.float32),
        pltpu.SemaphoreType.DMA,
        pltpu.SemaphoreType.DMA,
        pltpu.SemaphoreType.DMA,
        pltpu.SemaphoreType.DMA,
        pltpu.SemaphoreType.DMA,
        pltpu.SemaphoreType.DMA,
        pltpu.SemaphoreType.DMA,
        pltpu.SemaphoreType.DMA,
        pltpu.SemaphoreType.DMA,
        pltpu.SemaphoreType.DMA,
        pltpu.SemaphoreType.DMA,
        pltpu.SemaphoreType.DMA,
    ],
)
def _sc_gatherconv(table, idx, z_out, stats_out, idx_v,
                   a0, a1, b1, b2, b3, b4, b5, b6, b7, b8, stats_v,
                   as0, as1, s1, s2, s3, s4, s5, s6, s7, s8, ws0, ws1):
    cid = lax.axis_index("c")
    sid = lax.axis_index("s")
    wid = sid * NC + cid
    cnt = jnp.where(cid == 0, NCHUNK0, NCHUNK1)
    npair = cnt // 2
    chunk0 = jnp.where(cid == 0, NCHUNK0 * sid, NS * NCHUNK0 + NCHUNK1 * sid)
    base = chunk0 * CHUNK
    bufs = (b1, b2, b3, b4, b5, b6, b7, b8)
    bsems = (s1, s2, s3, s4, s5, s6, s7, s8)

    pltpu.sync_copy(idx.at[wid], idx_v)

    zeros = jnp.zeros((L,), jnp.float32)
    for g in range(NG):
        stats_v[0, pl.ds(g * L, L)] = zeros
        stats_v[1, pl.ds(g * L, L)] = zeros

    def src(c, k):
        return table.at[idx_v.at[k * cnt + c]]

    def fire(c, k, dst, sem):
        pltpu.async_copy(src(c, k), dst, sem)

    def wait_g(c, k, dst, sem):
        pltpu.make_async_copy(src(c, k), dst, sem).wait()

    def accum4(acc, bs):
        def row(r, carry):
            for g in range(NG):
                sl = pl.ds(g * L, L)
                v = acc[r, sl]
                for b in bs:
                    v = v + b[r, sl]
                acc[r, sl] = v
            return carry
        lax.fori_loop(0, CHUNK, row, 0)

    def epilogue(c, acc):
        row0 = base + c * CHUNK

        def epi_row(r, sums):
            valid = (row0 + r) < N
            out = []
            for g in range(NG):
                sl = pl.ds(g * L, L)
                v = acc[r, sl]
                zv = jnp.where(v >= 0.0, v, v * 0.01)
                acc[r, sl] = zv
                zm = jnp.where(valid, zv, 0.0)
                s0v, s1v = sums[g]
                out.append((s0v + zm, s1v + zm * zm))
            return tuple(out)

        sums = lax.fori_loop(0, CHUNK, epi_row,
                             tuple((zeros, zeros) for _ in range(NG)))
        for g in range(NG):
            sl = pl.ds(g * L, L)
            stats_v[0, sl] = stats_v[0, sl] + sums[g][0]
            stats_v[1, sl] = stats_v[1, sl] + sums[g][1]
        return row0

    fire(0, 0, a0, as0)
    for k in range(1, K):
        fire(0, k, bufs[k - 1], bsems[k - 1])

    def pair_body(j, carry):
        c0 = 2 * j
        wait_g(c0, 0, a0, as0)
        for k in (1, 2, 3, 4):
            wait_g(c0, k, bufs[k - 1], bsems[k - 1])
        accum4(a0, bufs[0:4])
        for k in (1, 2, 3, 4):
            fire(c0 + 1, k, bufs[k - 1], bsems[k - 1])
        for k in (5, 6, 7, 8):
            wait_g(c0, k, bufs[k - 1], bsems[k - 1])
        accum4(a0, bufs[4:8])
        for k in (5, 6, 7, 8):
            fire(c0 + 1, k, bufs[k - 1], bsems[k - 1])

        @pl.when(j > 0)
        def _():
            pltpu.make_async_copy(
                a1, z_out.at[pl.ds(base + (c0 - 1) * CHUNK, CHUNK)], ws1).wait()
        fire(c0 + 1, 0, a1, as1)
        row0 = epilogue(c0, a0)
        pltpu.async_copy(a0, z_out.at[pl.ds(row0, CHUNK)], ws0)

        c1 = 2 * j + 1
        wait_g(c1, 0, a1, as1)
        for k in (1, 2, 3, 4):
            wait_g(c1, k, bufs[k - 1], bsems[k - 1])
        accum4(a1, bufs[0:4])

        @pl.when(j < npair - 1)
        def _():
            for k in (1, 2, 3, 4):
                fire(c1 + 1, k, bufs[k - 1], bsems[k - 1])
        for k in (5, 6, 7, 8):
            wait_g(c1, k, bufs[k - 1], bsems[k - 1])
        accum4(a1, bufs[4:8])

        @pl.when(j < npair - 1)
        def _():
            for k in (5, 6, 7, 8):
                fire(c1 + 1, k, bufs[k - 1], bsems[k - 1])
            pltpu.make_async_copy(
                a0, z_out.at[pl.ds(base + c0 * CHUNK, CHUNK)], ws0).wait()
            fire(c1 + 1, 0, a0, as0)
        row1 = epilogue(c1, a1)
        pltpu.async_copy(a1, z_out.at[pl.ds(row1, CHUNK)], ws1)
        return carry

    lax.fori_loop(0, npair, pair_body, 0)

    pltpu.make_async_copy(
        a0, z_out.at[pl.ds(base + (cnt - 2) * CHUNK, CHUNK)], ws0).wait()
    pltpu.make_async_copy(
        a1, z_out.at[pl.ds(base + (cnt - 1) * CHUNK, CHUNK)], ws1).wait()
    pltpu.sync_copy(stats_v, stats_out.at[wid])


def _bn_coeffs(stats, gamma, beta):
    tot = jnp.sum(stats, axis=0)
    mean = tot[0:1, :] * (1.0 / N)
    ex2 = tot[1:2, :] * (1.0 / N)
    var = ex2 - mean * mean
    s = gamma * lax.rsqrt(var + EPS)
    t = beta - mean * s
    return s, t


def _mm_body(x_ref, w_ref, o_ref):
    r = jnp.dot(x_ref[...], w_ref[...], preferred_element_type=jnp.float32)
    for k in range(K):
        o_ref[k] = r[:, k * C:(k + 1) * C]


_mm_plain = pl.pallas_call(
    _mm_body,
    grid=(N_BLOCKS,),
    in_specs=[
        pl.BlockSpec((MM_BLK, C), lambda i: (i, 0)),
        pl.BlockSpec((C, K * C), lambda i: (0, 0)),
    ],
    out_specs=pl.BlockSpec((K, MM_BLK, C), lambda i: (0, i, 0)),
    out_shape=jax.ShapeDtypeStruct((K, N, C), jnp.float32),
)


def _mm_affine_body(z_ref, stats_ref, gb_ref, w_ref, o_ref):
    s, t = _bn_coeffs(stats_ref[...], gb_ref[0:1, :], gb_ref[1:2, :])
    zin = (z_ref[...] * s + t).astype(jnp.bfloat16)
    r = jnp.dot(zin, w_ref[...], preferred_element_type=jnp.float32)
    for k in range(K):
        o_ref[k] = r[:, k * C:(k + 1) * C]


_mm_affine = pl.pallas_call(
    _mm_affine_body,
    grid=(N_BLOCKS,),
    in_specs=[
        pl.BlockSpec((MM_BLK, C), lambda i: (i, 0)),
        pl.BlockSpec((NW, 2, C), lambda i: (0, 0, 0)),
        pl.BlockSpec((2, C), lambda i: (0, 0)),
        pl.BlockSpec((C, K * C), lambda i: (0, 0)),
    ],
    out_specs=pl.BlockSpec((K, MM_BLK, C), lambda i: (0, i, 0)),
    out_shape=jax.ShapeDtypeStruct((K, N, C), jnp.float32),
)


def _combine_body(z3_ref, z4_ref, st3_ref, gb3_ref, st4_ref, gb4_ref, o_ref):
    s3, t3 = _bn_coeffs(st3_ref[...], gb3_ref[0:1, :], gb3_ref[1:2, :])
    s4, t4 = _bn_coeffs(st4_ref[...], gb4_ref[0:1, :], gb4_ref[1:2, :])
    o_ref[...] = (z3_ref[...] * s3 + t3) + (z4_ref[...] * s4 + t4)


_combine = pl.pallas_call(
    _combine_body,
    grid=(N_BLOCKS,),
    in_specs=[
        pl.BlockSpec((MM_BLK, C), lambda i: (i, 0)),
        pl.BlockSpec((MM_BLK, C), lambda i: (i, 0)),
        pl.BlockSpec((NW, 2, C), lambda i: (0, 0, 0)),
        pl.BlockSpec((2, C), lambda i: (0, 0)),
        pl.BlockSpec((NW, 2, C), lambda i: (0, 0, 0)),
        pl.BlockSpec((2, C), lambda i: (0, 0)),
    ],
    out_specs=pl.BlockSpec((MM_BLK, C), lambda i: (i, 0)),
    out_shape=jax.ShapeDtypeStruct((N, C), jnp.float32),
)


def _prep_idx(nbr):
    taps = jnp.arange(K, dtype=jnp.int32)[:, None] * jnp.int32(N)
    idx = nbr + taps
    idx = jnp.concatenate(
        [idx, jnp.zeros((K, NPAD - N), jnp.int32)], axis=1)
    ch = idx.reshape(K, TOTCHUNK, CHUNK)
    n0 = NS * NCHUNK0
    c0 = ch[:, :n0].reshape(K, NS, NCHUNK0, CHUNK).transpose(1, 0, 2, 3)
    c0 = c0.reshape(NS, K * NCHUNK0, CHUNK)
    c1 = ch[:, n0:].reshape(K, NS, NCHUNK1, CHUNK).transpose(1, 0, 2, 3)
    c1 = c1.reshape(NS, K * NCHUNK1, CHUNK)
    c1 = jnp.concatenate(
        [c1, jnp.zeros((NS, IDXROWS - K * NCHUNK1, CHUNK), jnp.int32)], axis=1)
    return jnp.stack([c0, c1], axis=1).reshape(NW, IDXROWS, CHUNK)


def _wprep(w):
    return w.transpose(1, 0, 2).reshape(w.shape[1], K * C).astype(jnp.bfloat16)


def kernel(x, nbr_a, nbr_b, W1, W1_2, W2, W3,
           g0, b0, g0_2, b0_2, g1, b1, g2, b2):
    idx_a = _prep_idx(nbr_a)
    idx_b = _prep_idx(nbr_b)
    gb0 = jnp.stack([g0, b0])
    gb0_2 = jnp.stack([g0_2, b0_2])
    gb1 = jnp.stack([g1, b1])
    gb2 = jnp.stack([g2, b2])

    xb = x.astype(jnp.bfloat16)
    T1a = _mm_plain(xb, _wprep(W1))
    z1, st1 = _sc_gatherconv(T1a.reshape(K * N, C), idx_a)
    T1b = _mm_plain(xb, _wprep(W2))
    z2, st2 = _sc_gatherconv(T1b.reshape(K * N, C), idx_b)

    T2a = _mm_affine(z1, st1, gb0, _wprep(W1_2))
    T2b = _mm_affine(z2, st2, gb1, _wprep(W3))
    z3, st3 = _sc_gatherconv(T2a.reshape(K * N, C), idx_b)
    z4, st4 = _sc_gatherconv(T2b.reshape(K * N, C), idx_a)

    return _combine(z3, z4, st3, gb0_2, st4, gb2)


# speedup vs baseline: 1.0117x; 1.0117x over previous
import functools

import jax
import jax.numpy as jnp
from jax import lax
from jax.experimental import pallas as pl
from jax.experimental.pallas import tpu as pltpu
from jax.experimental.pallas import tpu_sc as plsc

N = 50000
C = 128
K = 9
EPS = 1e-5

NC = 2
NS = 16
NW = NC * NS
L = 16
NG = C // L

CHUNK = 56
NCHUNK0 = 28
NCHUNK1 = 28
TOTCHUNK = NS * (NCHUNK0 + NCHUNK1)
NPAD = CHUNK * TOTCHUNK
IDXROWS = 256

MM_BLK = 1000
N_BLOCKS = N // MM_BLK

_mesh = plsc.VectorSubcoreMesh(
    core_axis_name="c", subcore_axis_name="s", num_cores=NC, num_subcores=NS)


@functools.partial(
    pl.kernel,
    out_type=(
        jax.ShapeDtypeStruct((NPAD, C), jnp.float32),
        jax.ShapeDtypeStruct((NW, 2, C), jnp.float32),
    ),
    mesh=_mesh,
    scratch_types=[
        pltpu.VMEM((IDXROWS, CHUNK), jnp.int32),
        pltpu.VMEM((CHUNK, C), jnp.float32),
        pltpu.VMEM((CHUNK, C), jnp.float32),
        pltpu.VMEM((CHUNK, C), jnp.float32),
        pltpu.VMEM((CHUNK, C), jnp.float32),
        pltpu.VMEM((CHUNK, C), jnp.float32),
        pltpu.VMEM((CHUNK, C), jnp.float32),
        pltpu.VMEM((CHUNK, C), jnp.float32),
        pltpu.VMEM((CHUNK, C), jnp.float32),
        pltpu.VMEM((CHUNK, C), jnp.float32),
        pltpu.VMEM((CHUNK, C), jnp.float32),
        pltpu.VMEM((2, C), jnp.float32),
        pltpu.SemaphoreType.DMA,
        pltpu.SemaphoreType.DMA,
        pltpu.SemaphoreType.DMA,
        pltpu.SemaphoreType.DMA,
        pltpu.SemaphoreType.DMA,
        pltpu.SemaphoreType.DMA,
        pltpu.SemaphoreType.DMA,
        pltpu.SemaphoreType.DMA,
        pltpu.SemaphoreType.DMA,
        pltpu.SemaphoreType.DMA,
        pltpu.SemaphoreType.DMA,
        pltpu.SemaphoreType.DMA,
    ],
)
def _sc_gatherconv(table, idx, z_out, stats_out, idx_v,
                   a0, a1, b1, b2, b3, b4, b5, b6, b7, b8, stats_v,
                   as0, as1, s1, s2, s3, s4, s5, s6, s7, s8, ws0, ws1):
    cid = lax.axis_index("c")
    sid = lax.axis_index("s")
    wid = sid * NC + cid
    cnt = jnp.where(cid == 0, NCHUNK0, NCHUNK1)
    npair = cnt // 2
    chunk0 = jnp.where(cid == 0, NCHUNK0 * sid, NS * NCHUNK0 + NCHUNK1 * sid)
    base = chunk0 * CHUNK
    bufs = (b1, b2, b3, b4, b5, b6, b7, b8)
    bsems = (s1, s2, s3, s4, s5, s6, s7, s8)

    pltpu.sync_copy(idx.at[wid], idx_v)

    zeros = jnp.zeros((L,), jnp.float32)
    for g in range(NG):
        stats_v[0, pl.ds(g * L, L)] = zeros
        stats_v[1, pl.ds(g * L, L)] = zeros

    def src(c, k):
        return table.at[idx_v.at[k * cnt + c]]

    def fire(c, k, dst, sem):
        pltpu.async_copy(src(c, k), dst, sem)

    def wait_g(c, k, dst, sem):
        pltpu.make_async_copy(src(c, k), dst, sem).wait()

    def accum4(acc, bs):
        def row(r, carry):
            for g in range(NG):
                sl = pl.ds(g * L, L)
                v = acc[r, sl]
                for b in bs:
                    v = v + b[r, sl]
                acc[r, sl] = v
            return carry
        lax.fori_loop(0, CHUNK, row, 0)

    def epilogue(c, acc):
        row0 = base + c * CHUNK

        def epi_row(r, sums):
            valid = (row0 + r) < N
            out = []
            for g in range(NG):
                sl = pl.ds(g * L, L)
                v = acc[r, sl]
                zv = jnp.where(v >= 0.0, v, v * 0.01)
                acc[r, sl] = zv
                zm = jnp.where(valid, zv, 0.0)
                s0v, s1v = sums[g]
                out.append((s0v + zm, s1v + zm * zm))
            return tuple(out)

        sums = lax.fori_loop(0, CHUNK, epi_row,
                             tuple((zeros, zeros) for _ in range(NG)))
        for g in range(NG):
            sl = pl.ds(g * L, L)
            stats_v[0, sl] = stats_v[0, sl] + sums[g][0]
            stats_v[1, sl] = stats_v[1, sl] + sums[g][1]
        return row0

    fire(0, 0, a0, as0)
    for k in range(1, K):
        fire(0, k, bufs[k - 1], bsems[k - 1])

    def pair_body(j, carry):
        c0 = 2 * j
        wait_g(c0, 0, a0, as0)
        for k in (1, 2, 3, 4):
            wait_g(c0, k, bufs[k - 1], bsems[k - 1])
        accum4(a0, bufs[0:4])
        for k in (1, 2, 3, 4):
            fire(c0 + 1, k, bufs[k - 1], bsems[k - 1])
        for k in (5, 6, 7, 8):
            wait_g(c0, k, bufs[k - 1], bsems[k - 1])
        accum4(a0, bufs[4:8])
        for k in (5, 6, 7, 8):
            fire(c0 + 1, k, bufs[k - 1], bsems[k - 1])

        @pl.when(j > 0)
        def _():
            pltpu.make_async_copy(
                a1, z_out.at[pl.ds(base + (c0 - 1) * CHUNK, CHUNK)], ws1).wait()
        fire(c0 + 1, 0, a1, as1)
        row0 = epilogue(c0, a0)
        pltpu.async_copy(a0, z_out.at[pl.ds(row0, CHUNK)], ws0)

        c1 = 2 * j + 1
        wait_g(c1, 0, a1, as1)
        for k in (1, 2, 3, 4):
            wait_g(c1, k, bufs[k - 1], bsems[k - 1])
        accum4(a1, bufs[0:4])

        @pl.when(j < npair - 1)
        def _():
            for k in (1, 2, 3, 4):
                fire(c1 + 1, k, bufs[k - 1], bsems[k - 1])
        for k in (5, 6, 7, 8):
            wait_g(c1, k, bufs[k - 1], bsems[k - 1])
        accum4(a1, bufs[4:8])

        @pl.when(j < npair - 1)
        def _():
            for k in (5, 6, 7, 8):
                fire(c1 + 1, k, bufs[k - 1], bsems[k - 1])
            pltpu.make_async_copy(
                a0, z_out.at[pl.ds(base + c0 * CHUNK, CHUNK)], ws0).wait()
            fire(c1 + 1, 0, a0, as0)
        row1 = epilogue(c1, a1)
        pltpu.async_copy(a1, z_out.at[pl.ds(row1, CHUNK)], ws1)
        return carry

    lax.fori_loop(0, npair, pair_body, 0)

    pltpu.make_async_copy(
        a0, z_out.at[pl.ds(base + (cnt - 2) * CHUNK, CHUNK)], ws0).wait()
    pltpu.make_async_copy(
        a1, z_out.at[pl.ds(base + (cnt - 1) * CHUNK, CHUNK)], ws1).wait()
    pltpu.sync_copy(stats_v, stats_out.at[wid])


def _bn_coeffs(stats, gamma, beta):
    tot = jnp.sum(stats, axis=0)
    mean = tot[0:1, :] * (1.0 / N)
    ex2 = tot[1:2, :] * (1.0 / N)
    var = ex2 - mean * mean
    s = gamma * lax.rsqrt(var + EPS)
    t = beta - mean * s
    return s, t


def _mm2_body(x_ref, w1_ref, w2_ref, o1_ref, o2_ref):
    xb = x_ref[...]
    r1 = jnp.dot(xb, w1_ref[...], preferred_element_type=jnp.float32)
    r2 = jnp.dot(xb, w2_ref[...], preferred_element_type=jnp.float32)
    for k in range(K):
        o1_ref[k] = r1[:, k * C:(k + 1) * C]
        o2_ref[k] = r2[:, k * C:(k + 1) * C]


_mm2 = pl.pallas_call(
    _mm2_body,
    grid=(N_BLOCKS,),
    in_specs=[
        pl.BlockSpec((MM_BLK, C), lambda i: (i, 0)),
        pl.BlockSpec((C, K * C), lambda i: (0, 0)),
        pl.BlockSpec((C, K * C), lambda i: (0, 0)),
    ],
    out_specs=[
        pl.BlockSpec((K, MM_BLK, C), lambda i: (0, i, 0)),
        pl.BlockSpec((K, MM_BLK, C), lambda i: (0, i, 0)),
    ],
    out_shape=[
        jax.ShapeDtypeStruct((K, N, C), jnp.float32),
        jax.ShapeDtypeStruct((K, N, C), jnp.float32),
    ],
)


def _mm_affine_body(z_ref, stats_ref, gb_ref, w_ref, o_ref):
    s, t = _bn_coeffs(stats_ref[...], gb_ref[0:1, :], gb_ref[1:2, :])
    zin = (z_ref[...] * s + t).astype(jnp.bfloat16)
    r = jnp.dot(zin, w_ref[...], preferred_element_type=jnp.float32)
    for k in range(K):
        o_ref[k] = r[:, k * C:(k + 1) * C]


_mm_affine = pl.pallas_call(
    _mm_affine_body,
    grid=(N_BLOCKS,),
    in_specs=[
        pl.BlockSpec((MM_BLK, C), lambda i: (i, 0)),
        pl.BlockSpec((NW, 2, C), lambda i: (0, 0, 0)),
        pl.BlockSpec((2, C), lambda i: (0, 0)),
        pl.BlockSpec((C, K * C), lambda i: (0, 0)),
    ],
    out_specs=pl.BlockSpec((K, MM_BLK, C), lambda i: (0, i, 0)),
    out_shape=jax.ShapeDtypeStruct((K, N, C), jnp.float32),
)


def _combine_body(z3_ref, z4_ref, st3_ref, gb3_ref, st4_ref, gb4_ref, o_ref):
    s3, t3 = _bn_coeffs(st3_ref[...], gb3_ref[0:1, :], gb3_ref[1:2, :])
    s4, t4 = _bn_coeffs(st4_ref[...], gb4_ref[0:1, :], gb4_ref[1:2, :])
    o_ref[...] = (z3_ref[...] * s3 + t3) + (z4_ref[...] * s4 + t4)


_combine = pl.pallas_call(
    _combine_body,
    grid=(N_BLOCKS,),
    in_specs=[
        pl.BlockSpec((MM_BLK, C), lambda i: (i, 0)),
        pl.BlockSpec((MM_BLK, C), lambda i: (i, 0)),
        pl.BlockSpec((NW, 2, C), lambda i: (0, 0, 0)),
        pl.BlockSpec((2, C), lambda i: (0, 0)),
        pl.BlockSpec((NW, 2, C), lambda i: (0, 0, 0)),
        pl.BlockSpec((2, C), lambda i: (0, 0)),
    ],
    out_specs=pl.BlockSpec((MM_BLK, C), lambda i: (i, 0)),
    out_shape=jax.ShapeDtypeStruct((N, C), jnp.float32),
)


def _prep_idx(nbr):
    taps = jnp.arange(K, dtype=jnp.int32)[:, None] * jnp.int32(N)
    idx = nbr + taps
    idx = jnp.concatenate(
        [idx, jnp.zeros((K, NPAD - N), jnp.int32)], axis=1)
    ch = idx.reshape(K, TOTCHUNK, CHUNK)
    n0 = NS * NCHUNK0
    c0 = ch[:, :n0].reshape(K, NS, NCHUNK0, CHUNK).transpose(1, 0, 2, 3)
    c0 = c0.reshape(NS, K * NCHUNK0, CHUNK)
    c0 = jnp.concatenate(
        [c0, jnp.zeros((NS, IDXROWS - K * NCHUNK0, CHUNK), jnp.int32)], axis=1)
    c1 = ch[:, n0:].reshape(K, NS, NCHUNK1, CHUNK).transpose(1, 0, 2, 3)
    c1 = c1.reshape(NS, K * NCHUNK1, CHUNK)
    c1 = jnp.concatenate(
        [c1, jnp.zeros((NS, IDXROWS - K * NCHUNK1, CHUNK), jnp.int32)], axis=1)
    return jnp.stack([c0, c1], axis=1).reshape(NW, IDXROWS, CHUNK)


def _wprep(w):
    return w.transpose(1, 0, 2).reshape(w.shape[1], K * C).astype(jnp.bfloat16)


def kernel(x, nbr_a, nbr_b, W1, W1_2, W2, W3,
           g0, b0, g0_2, b0_2, g1, b1, g2, b2):
    idx_a = _prep_idx(nbr_a)
    idx_b = _prep_idx(nbr_b)
    gb0 = jnp.stack([g0, b0])
    gb0_2 = jnp.stack([g0_2, b0_2])
    gb1 = jnp.stack([g1, b1])
    gb2 = jnp.stack([g2, b2])

    xb = x.astype(jnp.bfloat16)
    T1a, T1b = _mm2(xb, _wprep(W1), _wprep(W2))
    z1, st1 = _sc_gatherconv(T1a.reshape(K * N, C), idx_a)
    z2, st2 = _sc_gatherconv(T1b.reshape(K * N, C), idx_b)

    T2a = _mm_affine(z1, st1, gb0, _wprep(W1_2))
    T2b = _mm_affine(z2, st2, gb1, _wprep(W3))
    z3, st3 = _sc_gatherconv(T2a.reshape(K * N, C), idx_b)
    z4, st4 = _sc_gatherconv(T2b.reshape(K * N, C), idx_a)

    return _combine(z3, z4, st3, gb0_2, st4, gb2)


# speedup vs baseline: 1.1075x; 1.0946x over previous
import functools

import jax
import jax.numpy as jnp
import numpy as np
from jax import lax
from jax.experimental import pallas as pl
from jax.experimental.pallas import tpu as pltpu
from jax.experimental.pallas import tpu_sc as plsc

N = 50000
C = 128
K = 9
EPS = 1e-5

NC = 2
NS = 16
NW = NC * NS
L = 16
NG = C // L

CHUNK = 56
NCHUNK = 28
NPAIR = NCHUNK // 2
RPW = CHUNK * NCHUNK
NPAD = RPW * NW
IDXROWS = 256

MM_BLK = 1000
N_BLOCKS = N // MM_BLK

_mesh = plsc.VectorSubcoreMesh(
    core_axis_name="c", subcore_axis_name="s", num_cores=NC, num_subcores=NS)


@functools.partial(
    pl.kernel,
    out_type=(
        jax.ShapeDtypeStruct((NPAD, C), jnp.float32),
        jax.ShapeDtypeStruct((NW, 2, C), jnp.float32),
    ),
    mesh=_mesh,
    scratch_types=[
        pltpu.VMEM((IDXROWS, CHUNK), jnp.int32),
        pltpu.VMEM((CHUNK, C), jnp.float32),
        pltpu.VMEM((CHUNK, C), jnp.float32),
        pltpu.VMEM((CHUNK, C), jnp.float32),
        pltpu.VMEM((CHUNK, C), jnp.float32),
        pltpu.VMEM((CHUNK, C), jnp.float32),
        pltpu.VMEM((CHUNK, C), jnp.float32),
        pltpu.VMEM((CHUNK, C), jnp.float32),
        pltpu.VMEM((CHUNK, C), jnp.float32),
        pltpu.VMEM((CHUNK, C), jnp.float32),
        pltpu.VMEM((CHUNK, C), jnp.float32),
        pltpu.VMEM((CHUNK, C), jnp.float32),
        pltpu.VMEM((2, C), jnp.float32),
        pltpu.SemaphoreType.DMA,
        pltpu.SemaphoreType.DMA,
        pltpu.SemaphoreType.DMA,
        pltpu.SemaphoreType.DMA,
        pltpu.SemaphoreType.DMA,
        pltpu.SemaphoreType.DMA,
        pltpu.SemaphoreType.DMA,
        pltpu.SemaphoreType.DMA,
        pltpu.SemaphoreType.DMA,
        pltpu.SemaphoreType.DMA,
        pltpu.SemaphoreType.DMA,
    ],
)
def _sc_gatherconv(table, idx, z_out, stats_out, idx_v,
                   a0, a1, b0, b1, b2, b3, b4, b5, b6, b7, b8, stats_v,
                   s0, s1, s2, s3, s4, s5, s6, s7, s8, ws0, ws1):
    wid = lax.axis_index("s") * NC + lax.axis_index("c")
    base = wid * RPW
    bufs = (b0, b1, b2, b3, b4, b5, b6, b7, b8)
    bsems = (s0, s1, s2, s3, s4, s5, s6, s7, s8)

    pltpu.sync_copy(idx.at[wid], idx_v)

    zeros = jnp.zeros((L,), jnp.float32)
    for g in range(NG):
        stats_v[0, pl.ds(g * L, L)] = zeros
        stats_v[1, pl.ds(g * L, L)] = zeros

    def src(c, k):
        return table.at[idx_v.at[k * NCHUNK + c]]

    def fire(c, k):
        pltpu.async_copy(src(c, k), bufs[k], bsems[k])

    def wait_g(c, k):
        pltpu.make_async_copy(src(c, k), bufs[k], bsems[k]).wait()

    def accum(acc, bs, first):
        def row(r, carry):
            for g in range(NG):
                sl = pl.ds(g * L, L)
                v = bs[0][r, sl]
                if not first:
                    v = v + acc[r, sl]
                for b in bs[1:]:
                    v = v + b[r, sl]
                acc[r, sl] = v
            return carry
        lax.fori_loop(0, CHUNK, row, 0)

    def epilogue(c, acc):
        row0 = base + c * CHUNK

        def epi_row(r, sums):
            valid = (row0 + r) < N
            out = []
            for g in range(NG):
                sl = pl.ds(g * L, L)
                v = acc[r, sl]
                zv = jnp.where(v >= 0.0, v, v * 0.01)
                acc[r, sl] = zv
                zm = jnp.where(valid, zv, 0.0)
                s0v, s1v = sums[g]
                out.append((s0v + zm, s1v + zm * zm))
            return tuple(out)

        sums = lax.fori_loop(0, CHUNK, epi_row,
                             tuple((zeros, zeros) for _ in range(NG)))
        for g in range(NG):
            sl = pl.ds(g * L, L)
            stats_v[0, sl] = stats_v[0, sl] + sums[g][0]
            stats_v[1, sl] = stats_v[1, sl] + sums[g][1]
        return row0

    for k in range(K):
        fire(0, k)

    def pair_body(j, carry):
        c0 = 2 * j
        for k in (0, 1, 2, 3, 4):
            wait_g(c0, k)

        @pl.when(j > 0)
        def _():
            pltpu.make_async_copy(
                a0, z_out.at[pl.ds(base + (c0 - 2) * CHUNK, CHUNK)], ws0).wait()
        accum(a0, bufs[0:5], first=True)
        for k in (0, 1, 2, 3, 4):
            fire(c0 + 1, k)
        for k in (5, 6, 7, 8):
            wait_g(c0, k)
        accum(a0, bufs[5:9], first=False)
        for k in (5, 6, 7, 8):
            fire(c0 + 1, k)
        row0 = epilogue(c0, a0)
        pltpu.async_copy(a0, z_out.at[pl.ds(row0, CHUNK)], ws0)

        c1 = 2 * j + 1
        for k in (0, 1, 2, 3, 4):
            wait_g(c1, k)

        @pl.when(j > 0)
        def _():
            pltpu.make_async_copy(
                a1, z_out.at[pl.ds(base + (c1 - 2) * CHUNK, CHUNK)], ws1).wait()
        accum(a1, bufs[0:5], first=True)

        @pl.when(j < NPAIR - 1)
        def _():
            for k in (0, 1, 2, 3, 4):
                fire(c1 + 1, k)
        for k in (5, 6, 7, 8):
            wait_g(c1, k)
        accum(a1, bufs[5:9], first=False)

        @pl.when(j < NPAIR - 1)
        def _():
            for k in (5, 6, 7, 8):
                fire(c1 + 1, k)
        row1 = epilogue(c1, a1)
        pltpu.async_copy(a1, z_out.at[pl.ds(row1, CHUNK)], ws1)
        return carry

    lax.fori_loop(0, NPAIR, pair_body, 0)

    pltpu.make_async_copy(
        a0, z_out.at[pl.ds(base + (NCHUNK - 2) * CHUNK, CHUNK)], ws0).wait()
    pltpu.make_async_copy(
        a1, z_out.at[pl.ds(base + (NCHUNK - 1) * CHUNK, CHUNK)], ws1).wait()
    pltpu.sync_copy(stats_v, stats_out.at[wid])


def _bn_coeffs(stats, gamma, beta):
    tot = jnp.sum(stats, axis=0)
    mean = tot[0:1, :] * (1.0 / N)
    ex2 = tot[1:2, :] * (1.0 / N)
    var = ex2 - mean * mean
    s = gamma * lax.rsqrt(var + EPS)
    t = beta - mean * s
    return s, t


def _mm2_body(x_ref, w1_ref, w2_ref, o1_ref, o2_ref):
    xb = x_ref[...]
    r1 = jnp.dot(xb, w1_ref[...], preferred_element_type=jnp.float32)
    r2 = jnp.dot(xb, w2_ref[...], preferred_element_type=jnp.float32)
    for k in range(K):
        o1_ref[k] = r1[:, k * C:(k + 1) * C]
        o2_ref[k] = r2[:, k * C:(k + 1) * C]


_mm2 = pl.pallas_call(
    _mm2_body,
    grid=(N_BLOCKS,),
    in_specs=[
        pl.BlockSpec((MM_BLK, C), lambda i: (i, 0)),
        pl.BlockSpec((C, K * C), lambda i: (0, 0)),
        pl.BlockSpec((C, K * C), lambda i: (0, 0)),
    ],
    out_specs=[
        pl.BlockSpec((K, MM_BLK, C), lambda i: (0, i, 0)),
        pl.BlockSpec((K, MM_BLK, C), lambda i: (0, i, 0)),
    ],
    out_shape=[
        jax.ShapeDtypeStruct((K, N, C), jnp.float32),
        jax.ShapeDtypeStruct((K, N, C), jnp.float32),
    ],
)


def _mm_affine_body(z_ref, stats_ref, gb_ref, w_ref, o_ref):
    s, t = _bn_coeffs(stats_ref[...], gb_ref[0:1, :], gb_ref[1:2, :])
    zin = (z_ref[...] * s + t).astype(jnp.bfloat16)
    r = jnp.dot(zin, w_ref[...], preferred_element_type=jnp.float32)
    for k in range(K):
        o_ref[k] = r[:, k * C:(k + 1) * C]


_mm_affine = pl.pallas_call(
    _mm_affine_body,
    grid=(N_BLOCKS,),
    in_specs=[
        pl.BlockSpec((MM_BLK, C), lambda i: (i, 0)),
        pl.BlockSpec((NW, 2, C), lambda i: (0, 0, 0)),
        pl.BlockSpec((2, C), lambda i: (0, 0)),
        pl.BlockSpec((C, K * C), lambda i: (0, 0)),
    ],
    out_specs=pl.BlockSpec((K, MM_BLK, C), lambda i: (0, i, 0)),
    out_shape=jax.ShapeDtypeStruct((K, N, C), jnp.float32),
)


def _combine_body(z3_ref, z4_ref, st3_ref, gb3_ref, st4_ref, gb4_ref, o_ref):
    s3, t3 = _bn_coeffs(st3_ref[...], gb3_ref[0:1, :], gb3_ref[1:2, :])
    s4, t4 = _bn_coeffs(st4_ref[...], gb4_ref[0:1, :], gb4_ref[1:2, :])
    o_ref[...] = (z3_ref[...] * s3 + t3) + (z4_ref[...] * s4 + t4)


_combine = pl.pallas_call(
    _combine_body,
    grid=(N_BLOCKS,),
    in_specs=[
        pl.BlockSpec((MM_BLK, C), lambda i: (i, 0)),
        pl.BlockSpec((MM_BLK, C), lambda i: (i, 0)),
        pl.BlockSpec((NW, 2, C), lambda i: (0, 0, 0)),
        pl.BlockSpec((2, C), lambda i: (0, 0)),
        pl.BlockSpec((NW, 2, C), lambda i: (0, 0, 0)),
        pl.BlockSpec((2, C), lambda i: (0, 0)),
    ],
    out_specs=pl.BlockSpec((MM_BLK, C), lambda i: (i, 0)),
    out_shape=jax.ShapeDtypeStruct((N, C), jnp.float32),
)


def _prep_idx(nbr):
    taps = jnp.arange(K, dtype=jnp.int32)[:, None] * jnp.int32(N)
    idx = nbr + taps
    idx = jnp.concatenate(
        [idx, jnp.zeros((K, NPAD - N), jnp.int32)], axis=1)
    idx = idx.reshape(K, NW, NCHUNK, CHUNK).transpose(1, 0, 2, 3)
    idx = idx.reshape(NW, K * NCHUNK, CHUNK)
    return jnp.concatenate(
        [idx, jnp.zeros((NW, IDXROWS - K * NCHUNK, CHUNK), jnp.int32)], axis=1)


def _wprep(w):
    return w.transpose(1, 0, 2).reshape(w.shape[1], K * C).astype(jnp.bfloat16)


def kernel(x, nbr_a, nbr_b, W1, W1_2, W2, W3,
           g0, b0, g0_2, b0_2, g1, b1, g2, b2):
    idx_a = _prep_idx(nbr_a)
    idx_b = _prep_idx(nbr_b)
    gb0 = jnp.stack([g0, b0])
    gb0_2 = jnp.stack([g0_2, b0_2])
    gb1 = jnp.stack([g1, b1])
    gb2 = jnp.stack([g2, b2])

    xb = x.astype(jnp.bfloat16)
    T1a, T1b = _mm2(xb, _wprep(W1), _wprep(W2))
    z1, st1 = _sc_gatherconv(T1a.reshape(K * N, C), idx_a)
    z2, st2 = _sc_gatherconv(T1b.reshape(K * N, C), idx_b)

    T2a = _mm_affine(z1, st1, gb0, _wprep(W1_2))
    T2b = _mm_affine(z2, st2, gb1, _wprep(W3))
    z3, st3 = _sc_gatherconv(T2a.reshape(K * N, C), idx_b)
    z4, st4 = _sc_gatherconv(T2b.reshape(K * N, C), idx_a)

    return _combine(z3, z4, st3, gb0_2, st4, gb2)


# speedup vs baseline: 1.1225x; 1.0136x over previous
import functools

import jax
import jax.numpy as jnp
import numpy as np
from jax import lax
from jax.experimental import pallas as pl
from jax.experimental.pallas import tpu as pltpu
from jax.experimental.pallas import tpu_sc as plsc

N = 50000
C = 128
K = 9
EPS = 1e-5

NC = 2
NS = 16
NW = NC * NS
L = 16
NG = C // L

CHUNK = 56
NCHUNK = 28
NPAIR = NCHUNK // 2
RPW = CHUNK * NCHUNK
NPAD = RPW * NW
IDXROWS = 256

MM_BLK = 2000
N_BLOCKS = N // MM_BLK

_mesh = plsc.VectorSubcoreMesh(
    core_axis_name="c", subcore_axis_name="s", num_cores=NC, num_subcores=NS)


@functools.partial(
    pl.kernel,
    out_type=(
        jax.ShapeDtypeStruct((NPAD, C), jnp.float32),
        jax.ShapeDtypeStruct((NW, 2, C), jnp.float32),
    ),
    mesh=_mesh,
    scratch_types=[
        pltpu.VMEM((IDXROWS, CHUNK), jnp.int32),
        pltpu.VMEM((CHUNK, C), jnp.float32),
        pltpu.VMEM((CHUNK, C), jnp.float32),
        pltpu.VMEM((CHUNK, C), jnp.float32),
        pltpu.VMEM((CHUNK, C), jnp.float32),
        pltpu.VMEM((CHUNK, C), jnp.float32),
        pltpu.VMEM((CHUNK, C), jnp.float32),
        pltpu.VMEM((CHUNK, C), jnp.float32),
        pltpu.VMEM((CHUNK, C), jnp.float32),
        pltpu.VMEM((CHUNK, C), jnp.float32),
        pltpu.VMEM((CHUNK, C), jnp.float32),
        pltpu.VMEM((CHUNK, C), jnp.float32),
        pltpu.VMEM((2, C), jnp.float32),
        pltpu.SemaphoreType.DMA,
        pltpu.SemaphoreType.DMA,
        pltpu.SemaphoreType.DMA,
        pltpu.SemaphoreType.DMA,
        pltpu.SemaphoreType.DMA,
        pltpu.SemaphoreType.DMA,
        pltpu.SemaphoreType.DMA,
        pltpu.SemaphoreType.DMA,
        pltpu.SemaphoreType.DMA,
        pltpu.SemaphoreType.DMA,
        pltpu.SemaphoreType.DMA,
    ],
)
def _sc_gatherconv(table, idx, z_out, stats_out, idx_v,
                   a0, a1, b0, b1, b2, b3, b4, b5, b6, b7, b8, stats_v,
                   s0, s1, s2, s3, s4, s5, s6, s7, s8, ws0, ws1):
    wid = lax.axis_index("s") * NC + lax.axis_index("c")
    base = wid * RPW
    bufs = (b0, b1, b2, b3, b4, b5, b6, b7, b8)
    bsems = (s0, s1, s2, s3, s4, s5, s6, s7, s8)

    pltpu.sync_copy(idx.at[wid], idx_v)

    zeros = jnp.zeros((L,), jnp.float32)
    for g in range(NG):
        stats_v[0, pl.ds(g * L, L)] = zeros
        stats_v[1, pl.ds(g * L, L)] = zeros

    def src(c, k):
        return table.at[idx_v.at[k * NCHUNK + c]]

    def fire(c, k):
        pltpu.async_copy(src(c, k), bufs[k], bsems[k])

    def wait_g(c, k):
        pltpu.make_async_copy(src(c, k), bufs[k], bsems[k]).wait()

    def accum(acc, bs, first):
        def row(r, carry):
            for g in range(NG):
                sl = pl.ds(g * L, L)
                v = bs[0][r, sl]
                if not first:
                    v = v + acc[r, sl]
                for b in bs[1:]:
                    v = v + b[r, sl]
                acc[r, sl] = v
            return carry
        lax.fori_loop(0, CHUNK, row, 0)

    def epilogue(c, acc):
        row0 = base + c * CHUNK

        def epi_row(r, sums):
            valid = (row0 + r) < N
            out = []
            for g in range(NG):
                sl = pl.ds(g * L, L)
                v = acc[r, sl]
                zv = jnp.where(v >= 0.0, v, v * 0.01)
                acc[r, sl] = zv
                zm = jnp.where(valid, zv, 0.0)
                s0v, s1v = sums[g]
                out.append((s0v + zm, s1v + zm * zm))
            return tuple(out)

        sums = lax.fori_loop(0, CHUNK, epi_row,
                             tuple((zeros, zeros) for _ in range(NG)))
        for g in range(NG):
            sl = pl.ds(g * L, L)
            stats_v[0, sl] = stats_v[0, sl] + sums[g][0]
            stats_v[1, sl] = stats_v[1, sl] + sums[g][1]
        return row0

    for k in range(K):
        fire(0, k)

    def pair_body(j, carry):
        c0 = 2 * j
        for k in (0, 1, 2, 3, 4):
            wait_g(c0, k)

        @pl.when(j > 0)
        def _():
            pltpu.make_async_copy(
                a0, z_out.at[pl.ds(base + (c0 - 2) * CHUNK, CHUNK)], ws0).wait()
        accum(a0, bufs[0:5], first=True)
        for k in (0, 1, 2, 3, 4):
            fire(c0 + 1, k)
        for k in (5, 6, 7, 8):
            wait_g(c0, k)
        accum(a0, bufs[5:9], first=False)
        for k in (5, 6, 7, 8):
            fire(c0 + 1, k)
        row0 = epilogue(c0, a0)
        pltpu.async_copy(a0, z_out.at[pl.ds(row0, CHUNK)], ws0)

        c1 = 2 * j + 1
        for k in (0, 1, 2, 3, 4):
            wait_g(c1, k)

        @pl.when(j > 0)
        def _():
            pltpu.make_async_copy(
                a1, z_out.at[pl.ds(base + (c1 - 2) * CHUNK, CHUNK)], ws1).wait()
        accum(a1, bufs[0:5], first=True)

        @pl.when(j < NPAIR - 1)
        def _():
            for k in (0, 1, 2, 3, 4):
                fire(c1 + 1, k)
        for k in (5, 6, 7, 8):
            wait_g(c1, k)
        accum(a1, bufs[5:9], first=False)

        @pl.when(j < NPAIR - 1)
        def _():
            for k in (5, 6, 7, 8):
                fire(c1 + 1, k)
        row1 = epilogue(c1, a1)
        pltpu.async_copy(a1, z_out.at[pl.ds(row1, CHUNK)], ws1)
        return carry

    lax.fori_loop(0, NPAIR, pair_body, 0)

    pltpu.make_async_copy(
        a0, z_out.at[pl.ds(base + (NCHUNK - 2) * CHUNK, CHUNK)], ws0).wait()
    pltpu.make_async_copy(
        a1, z_out.at[pl.ds(base + (NCHUNK - 1) * CHUNK, CHUNK)], ws1).wait()
    pltpu.sync_copy(stats_v, stats_out.at[wid])


def _bn_coeffs(stats, gamma, beta):
    tot = jnp.sum(stats, axis=0)
    mean = tot[0:1, :] * (1.0 / N)
    ex2 = tot[1:2, :] * (1.0 / N)
    var = ex2 - mean * mean
    s = gamma * lax.rsqrt(var + EPS)
    t = beta - mean * s
    return s, t


def _mm2_body(x_ref, w1_ref, w2_ref, o1_ref, o2_ref):
    xb = x_ref[...]
    r1 = jnp.dot(xb, w1_ref[...], preferred_element_type=jnp.float32)
    r2 = jnp.dot(xb, w2_ref[...], preferred_element_type=jnp.float32)
    for k in range(K):
        o1_ref[k] = r1[:, k * C:(k + 1) * C]
        o2_ref[k] = r2[:, k * C:(k + 1) * C]


_mm2 = pl.pallas_call(
    _mm2_body,
    grid=(N_BLOCKS,),
    in_specs=[
        pl.BlockSpec((MM_BLK, C), lambda i: (i, 0)),
        pl.BlockSpec((C, K * C), lambda i: (0, 0)),
        pl.BlockSpec((C, K * C), lambda i: (0, 0)),
    ],
    out_specs=[
        pl.BlockSpec((K, MM_BLK, C), lambda i: (0, i, 0)),
        pl.BlockSpec((K, MM_BLK, C), lambda i: (0, i, 0)),
    ],
    out_shape=[
        jax.ShapeDtypeStruct((K, N, C), jnp.float32),
        jax.ShapeDtypeStruct((K, N, C), jnp.float32),
    ],
)


def _mm_affine_body(z_ref, stats_ref, gb_ref, w_ref, o_ref):
    s, t = _bn_coeffs(stats_ref[...], gb_ref[0:1, :], gb_ref[1:2, :])
    zin = (z_ref[...] * s + t).astype(jnp.bfloat16)
    r = jnp.dot(zin, w_ref[...], preferred_element_type=jnp.float32)
    for k in range(K):
        o_ref[k] = r[:, k * C:(k + 1) * C]


_mm_affine = pl.pallas_call(
    _mm_affine_body,
    grid=(N_BLOCKS,),
    in_specs=[
        pl.BlockSpec((MM_BLK, C), lambda i: (i, 0)),
        pl.BlockSpec((NW, 2, C), lambda i: (0, 0, 0)),
        pl.BlockSpec((2, C), lambda i: (0, 0)),
        pl.BlockSpec((C, K * C), lambda i: (0, 0)),
    ],
    out_specs=pl.BlockSpec((K, MM_BLK, C), lambda i: (0, i, 0)),
    out_shape=jax.ShapeDtypeStruct((K, N, C), jnp.float32),
)


def _combine_body(z3_ref, z4_ref, st3_ref, gb3_ref, st4_ref, gb4_ref, o_ref):
    s3, t3 = _bn_coeffs(st3_ref[...], gb3_ref[0:1, :], gb3_ref[1:2, :])
    s4, t4 = _bn_coeffs(st4_ref[...], gb4_ref[0:1, :], gb4_ref[1:2, :])
    o_ref[...] = (z3_ref[...] * s3 + t3) + (z4_ref[...] * s4 + t4)


_combine = pl.pallas_call(
    _combine_body,
    grid=(N_BLOCKS,),
    in_specs=[
        pl.BlockSpec((MM_BLK, C), lambda i: (i, 0)),
        pl.BlockSpec((MM_BLK, C), lambda i: (i, 0)),
        pl.BlockSpec((NW, 2, C), lambda i: (0, 0, 0)),
        pl.BlockSpec((2, C), lambda i: (0, 0)),
        pl.BlockSpec((NW, 2, C), lambda i: (0, 0, 0)),
        pl.BlockSpec((2, C), lambda i: (0, 0)),
    ],
    out_specs=pl.BlockSpec((MM_BLK, C), lambda i: (i, 0)),
    out_shape=jax.ShapeDtypeStruct((N, C), jnp.float32),
)


def _prep_idx(nbr):
    taps = jnp.arange(K, dtype=jnp.int32)[:, None] * jnp.int32(N)
    idx = nbr + taps
    idx = jnp.concatenate(
        [idx, jnp.zeros((K, NPAD - N), jnp.int32)], axis=1)
    idx = idx.reshape(K, NW, NCHUNK, CHUNK).transpose(1, 0, 2, 3)
    idx = idx.reshape(NW, K * NCHUNK, CHUNK)
    return jnp.concatenate(
        [idx, jnp.zeros((NW, IDXROWS - K * NCHUNK, CHUNK), jnp.int32)], axis=1)


def _wprep(w):
    return w.transpose(1, 0, 2).reshape(w.shape[1], K * C).astype(jnp.bfloat16)


def kernel(x, nbr_a, nbr_b, W1, W1_2, W2, W3,
           g0, b0, g0_2, b0_2, g1, b1, g2, b2):
    idx_a = _prep_idx(nbr_a)
    idx_b = _prep_idx(nbr_b)
    gb0 = jnp.stack([g0, b0])
    gb0_2 = jnp.stack([g0_2, b0_2])
    gb1 = jnp.stack([g1, b1])
    gb2 = jnp.stack([g2, b2])

    xb = x.astype(jnp.bfloat16)
    T1a, T1b = _mm2(xb, _wprep(W1), _wprep(W2))
    z1, st1 = _sc_gatherconv(T1a.reshape(K * N, C), idx_a)
    z2, st2 = _sc_gatherconv(T1b.reshape(K * N, C), idx_b)

    T2a = _mm_affine(z1, st1, gb0, _wprep(W1_2))
    T2b = _mm_affine(z2, st2, gb1, _wprep(W3))
    z3, st3 = _sc_gatherconv(T2a.reshape(K * N, C), idx_b)
    z4, st4 = _sc_gatherconv(T2b.reshape(K * N, C), idx_a)

    return _combine(z3, z4, st3, gb0_2, st4, gb2)
